# baseline (speedup 1.0000x reference)
import functools

import jax
import jax.numpy as jnp
from jax import lax
from jax.experimental import pallas as pl
from jax.experimental.pallas import tpu as pltpu
from jax.experimental.pallas import tpu_sc as plsc

F32 = jnp.float32
LANES = 16
NCORE = 2
NSUB = 16
NW = NCORE * NSUB
BATCH = 128

_SC_PARAMS = pltpu.CompilerParams(use_tc_tiling_on_sc=False)


def _leaky_relu(v):
    return jnp.where(v >= 0, v, 0.2 * v)


def _w_from_rows(gs_i, gd_i):
    return jnp.exp(_leaky_relu(gs_i + jnp.flip(gd_i)))


def _mm1_body(x_ref, w1_ref, ac_ref, xwlo_ref, xwhi_ref, a1_ref):
    xw = jnp.dot(x_ref[...], w1_ref[...], preferred_element_type=F32)
    xwlo_ref[...] = xw[:, :128]
    xwhi_ref[...] = xw[:, 128:]
    a1_ref[...] = jnp.dot(xw, ac_ref[...], preferred_element_type=F32)


def _mid_body(plo_ref, phi_ref, asum_ref, b1_ref, w2_ref, a2c_ref,
              xw2_ref, a2_ref):
    lo = plo_ref[0] + plo_ref[1]
    hi = phi_ref[0] + phi_ref[1]
    s = asum_ref[0] + asum_ref[1]
    s = jnp.where(s == 0, 1.0, s)
    blk = lo.shape[0]
    den_lo = jnp.broadcast_to(s[:, 0:4, None], (blk, 4, 32)).reshape(blk, 128)
    den_hi = jnp.broadcast_to(s[:, 4:8, None], (blk, 4, 32)).reshape(blk, 128)
    h = jnp.concatenate([lo / den_lo, hi / den_hi], axis=1) + b1_ref[...]
    h = jnp.where(h > 0, h, jnp.exp(jnp.minimum(h, 0.0)) - 1.0)
    xw2 = jnp.dot(h, w2_ref[...], preferred_element_type=F32)
    xw2_ref[...] = xw2
    a2_ref[...] = jnp.dot(xw2, a2c_ref[...], preferred_element_type=F32)


def _fin_body(p2_ref, asum2_ref, b2_ref, out_ref):
    o = p2_ref[0] + p2_ref[1]
    s = asum2_ref[0, :, 0:1] + asum2_ref[1, :, 0:1]
    s = jnp.where(s == 0, 1.0, s)
    o = o[:, :40] / s + b2_ref[...]
    m = jnp.max(o, axis=1, keepdims=True)
    lse = jnp.log(jnp.sum(jnp.exp(o - m), axis=1, keepdims=True))
    out_ref[...] = o - m - lse


def _zero_rows(buf, nrow, ncol16):
    zero = jnp.zeros((LANES,), F32)

    def body(i, _):
        for k in range(ncol16):
            buf[i, pl.ds(LANES * k, LANES)] = zero
        return 0

    lax.fori_loop(0, nrow, body, 0)


def _wloop(gs, gd, wb):
    def body(i, _):
        wb[i, :] = _w_from_rows(gs[i, :], gd[i, :])
        return 0

    lax.fori_loop(0, BATCH, body, 0)


def _mloop(wb, xr, h0):
    def body(e, _):
        wvec = wb[e, :]
        for v in range(8):
            wv = wvec[h0 + (v // 2)]
            xr[e, pl.ds(LANES * v, LANES)] = xr[e, pl.ds(LANES * v, LANES)] * wv
        return 0

    lax.fori_loop(0, BATCH, body, 0)


CHUNK = 8


def _make_feat1_kernel(n_pad, nb, nb_pad):
    rows_per_tile = n_pad // NSUB
    nch = nb_pad // CHUNK

    @functools.partial(
        pl.kernel,
        out_type=(
            jax.ShapeDtypeStruct((NCORE, n_pad, 16), F32),
            jax.ShapeDtypeStruct((NCORE, n_pad, 128), F32),
            jax.ShapeDtypeStruct((NCORE, n_pad, 128), F32),
            jax.ShapeDtypeStruct((NW, nb_pad * BATCH, 16), F32),
        ),
        mesh=plsc.VectorSubcoreMesh(core_axis_name="c", subcore_axis_name="s"),
        compiler_params=_SC_PARAMS,
        scratch_types=[
            pltpu.VMEM((CHUNK, BATCH), jnp.int32),
            pltpu.VMEM((CHUNK, BATCH), jnp.int32),
            pltpu.VMEM((BATCH, 16), F32),
            pltpu.VMEM((BATCH, 16), F32),
            pltpu.VMEM((BATCH, 16), F32),
            pltpu.VMEM((BATCH, 128), F32),
            pltpu.VMEM_SHARED((n_pad, 128), F32),
            pltpu.VMEM_SHARED((n_pad, 16), F32),
            pltpu.SemaphoreType.DMA,
            pltpu.SemaphoreType.DMA,
            pltpu.SemaphoreType.DMA,
        ],
    )
    def feat1_k(src_hbm, dst_hbm, a_hbm, xwlo_hbm, xwhi_hbm,
                asum_hbm, plo_hbm, phi_hbm, wdump_hbm,
                src_c, dst_c, gs, gd, wb, xr, acc, accs, sem0, sem1, sem2):
        c = lax.axis_index("c")
        s = lax.axis_index("s")
        wid = s * NCORE + c
        row0 = s * rows_per_tile
        rows = pl.ds(row0, rows_per_tile)

        def walk(body_fn):
            def chunk_body(ch, _):
                pltpu.sync_copy(src_hbm.at[wid, pl.ds(ch * CHUNK, CHUNK)],
                                src_c)
                pltpu.sync_copy(dst_hbm.at[wid, pl.ds(ch * CHUNK, CHUNK)],
                                dst_c)

                def inner(j, _):
                    body_fn(ch * CHUNK + j, j)
                    return 0

                lax.fori_loop(0, CHUNK, inner, 0)
                return 0

            lax.fori_loop(0, nch, chunk_body, 0)

        _zero_rows(gs, BATCH, 1)
        _zero_rows(xr, BATCH, 8)
        for i in range(rows_per_tile // BATCH):
            pltpu.sync_copy(gs, accs.at[pl.ds(row0 + i * BATCH, BATCH)])
            pltpu.sync_copy(xr, acc.at[pl.ds(row0 + i * BATCH, BATCH)])
        plsc.subcore_barrier()

        def batch1(b, j):
            cp0 = pltpu.async_copy(a_hbm.at[src_c.at[j]], gs, sem0)
            cp1 = pltpu.async_copy(a_hbm.at[dst_c.at[j]], gd, sem1)
            cp2 = pltpu.async_copy(xwlo_hbm.at[src_c.at[j]], xr, sem2)
            cp0.wait()
            cp1.wait()
            _wloop(gs, gd, wb)
            cpw = pltpu.async_copy(
                wb, wdump_hbm.at[wid, pl.ds(b * BATCH, BATCH)], sem0)
            pltpu.sync_copy(wb, accs.at[dst_c.at[j]], add=True)
            cp2.wait()
            _mloop(wb, xr, 0)
            pltpu.sync_copy(xr, acc.at[dst_c.at[j]], add=True)
            cpw.wait()

        walk(batch1)
        plsc.subcore_barrier()
        pltpu.sync_copy(accs.at[rows], asum_hbm.at[c, rows])
        pltpu.sync_copy(acc.at[rows], plo_hbm.at[c, rows])
        plsc.subcore_barrier()
        _zero_rows(xr, BATCH, 8)
        for i in range(rows_per_tile // BATCH):
            pltpu.sync_copy(xr, acc.at[pl.ds(row0 + i * BATCH, BATCH)])
        plsc.subcore_barrier()

        def batch2(b, j):
            cpw = pltpu.async_copy(
                wdump_hbm.at[wid, pl.ds(b * BATCH, BATCH)], wb, sem1)
            cp2 = pltpu.async_copy(xwhi_hbm.at[src_c.at[j]], xr, sem2)
            cpw.wait()
            cp2.wait()
            _mloop(wb, xr, 4)
            pltpu.sync_copy(xr, acc.at[dst_c.at[j]], add=True)

        walk(batch2)
        plsc.subcore_barrier()
        pltpu.sync_copy(acc.at[rows], phi_hbm.at[c, rows])

    return feat1_k


def _make_feat2_kernel(n_pad, nb, nb_pad):
    rows_per_tile = n_pad // NSUB
    nch = nb_pad // CHUNK

    @functools.partial(
        pl.kernel,
        out_type=(
            jax.ShapeDtypeStruct((NCORE, n_pad, 16), F32),
            jax.ShapeDtypeStruct((NCORE, n_pad, 128), F32),
        ),
        mesh=plsc.VectorSubcoreMesh(core_axis_name="c", subcore_axis_name="s"),
        compiler_params=_SC_PARAMS,
        scratch_types=[
            pltpu.VMEM((CHUNK, BATCH), jnp.int32),
            pltpu.VMEM((CHUNK, BATCH), jnp.int32),
            pltpu.VMEM((BATCH, 16), F32),
            pltpu.VMEM((BATCH, 16), F32),
            pltpu.VMEM((BATCH, 16), F32),
            pltpu.VMEM((BATCH, 128), F32),
            pltpu.VMEM_SHARED((n_pad, 128), F32),
            pltpu.VMEM_SHARED((n_pad, 16), F32),
            pltpu.SemaphoreType.DMA,
            pltpu.SemaphoreType.DMA,
            pltpu.SemaphoreType.DMA,
        ],
    )
    def feat2_k(src_hbm, dst_hbm, a_hbm, xw_hbm, asum_hbm, out_hbm,
                src_c, dst_c, gs, gd, wb, xr, acc, accs, sem0, sem1, sem2):
        c = lax.axis_index("c")
        s = lax.axis_index("s")
        wid = s * NCORE + c
        row0 = s * rows_per_tile
        rows = pl.ds(row0, rows_per_tile)

        _zero_rows(gs, BATCH, 1)
        _zero_rows(xr, BATCH, 8)
        for i in range(rows_per_tile // BATCH):
            pltpu.sync_copy(gs, accs.at[pl.ds(row0 + i * BATCH, BATCH)])
            pltpu.sync_copy(xr, acc.at[pl.ds(row0 + i * BATCH, BATCH)])
        plsc.subcore_barrier()

        def batch_body(b, j):
            cp0 = pltpu.async_copy(a_hbm.at[src_c.at[j]], gs, sem0)
            cp1 = pltpu.async_copy(a_hbm.at[dst_c.at[j]], gd, sem1)
            cp2 = pltpu.async_copy(xw_hbm.at[src_c.at[j]], xr, sem2)
            cp0.wait()
            cp1.wait()
            _wloop(gs, gd, wb)
            pltpu.sync_copy(wb, accs.at[dst_c.at[j]], add=True)
            cp2.wait()
            _mloop(wb, xr, 0)
            pltpu.sync_copy(xr, acc.at[dst_c.at[j]], add=True)

        def chunk_body(ch, _):
            pltpu.sync_copy(src_hbm.at[wid, pl.ds(ch * CHUNK, CHUNK)], src_c)
            pltpu.sync_copy(dst_hbm.at[wid, pl.ds(ch * CHUNK, CHUNK)], dst_c)

            def inner(j, _):
                batch_body(ch * CHUNK + j, j)
                return 0

            lax.fori_loop(0, CHUNK, inner, 0)
            return 0

        lax.fori_loop(0, nch, chunk_body, 0)
        plsc.subcore_barrier()
        pltpu.sync_copy(accs.at[rows], asum_hbm.at[c, rows])
        pltpu.sync_copy(acc.at[rows], out_hbm.at[c, rows])

    return feat2_k


def kernel(x, edge_index, W1, att_src1, att_dst1, b1, W2, att_src2, att_dst2, b2):
    N, F = x.shape
    H1, C1 = att_src1.shape[1], att_src1.shape[2]
    D1 = H1 * C1
    NC = W2.shape[1]
    D2P = 128
    E = edge_index.shape[1]
    E2 = E + N
    n_pad = -(-(N + 1) // 2048) * 2048
    nb = -(-E2 // (NW * BATCH))
    nb_pad = -(-nb // 8) * 8
    e_pad = NW * nb * BATCH

    loop = jnp.arange(N, dtype=jnp.int32)
    src = jnp.concatenate([edge_index[0].astype(jnp.int32), loop])
    dst = jnp.concatenate([edge_index[1].astype(jnp.int32), loop])
    pad = jnp.full((e_pad - E2,), N, jnp.int32)
    src_p = jnp.pad(jnp.concatenate([src, pad]).reshape(NW, nb, BATCH),
                    ((0, 0), (0, nb_pad - nb), (0, 0)), constant_values=N)
    dst_p = jnp.pad(jnp.concatenate([dst, pad]).reshape(NW, nb, BATCH),
                    ((0, 0), (0, nb_pad - nb), (0, 0)), constant_values=N)

    x_pad = jnp.pad(x, ((0, n_pad - N), (0, 0)))
    eye1 = jnp.eye(H1, dtype=F32)
    a_src_m = (att_src1[0][:, :, None] * eye1[:, None, :]).reshape(D1, H1)
    a_dst_m = (att_dst1[0][:, :, None] * eye1[:, None, :]).reshape(D1, H1)
    a_comb_m = jnp.concatenate([a_src_m, a_dst_m[:, ::-1]], axis=1)
    w2_p = jnp.pad(W2, ((0, 0), (0, D2P - NC)))
    a2s_m = jnp.pad(jnp.tile(att_src2[0, 0][:, None], (1, 8)),
                    ((0, D2P - NC), (0, 0)))
    a2d_m = jnp.pad(jnp.tile(att_dst2[0, 0][:, None], (1, 8)),
                    ((0, D2P - NC), (0, 0)))
    a2_comb_m = jnp.concatenate([a2s_m, a2d_m], axis=1)
    b1_2d = b1[None, :]
    b2_2d = b2[None, :]

    BLK = 512
    grid = (n_pad // BLK,)

    xwlo, xwhi, a1 = pl.pallas_call(
        _mm1_body,
        grid=grid,
        in_specs=[
            pl.BlockSpec((BLK, F), lambda i: (i, 0)),
            pl.BlockSpec((F, D1), lambda i: (0, 0)),
            pl.BlockSpec((D1, 16), lambda i: (0, 0)),
        ],
        out_specs=[
            pl.BlockSpec((BLK, 128), lambda i: (i, 0)),
            pl.BlockSpec((BLK, 128), lambda i: (i, 0)),
            pl.BlockSpec((BLK, 16), lambda i: (i, 0)),
        ],
        out_shape=[
            jax.ShapeDtypeStruct((n_pad, 128), F32),
            jax.ShapeDtypeStruct((n_pad, 128), F32),
            jax.ShapeDtypeStruct((n_pad, 16), F32),
        ],
    )(x_pad, W1, a_comb_m)

    asum_p, plo, phi, _ = _make_feat1_kernel(n_pad, nb, nb_pad)(
        src_p, dst_p, a1, xwlo, xwhi)

    xw2, a2 = pl.pallas_call(
        _mid_body,
        grid=grid,
        in_specs=[
            pl.BlockSpec((NCORE, BLK, 128), lambda i: (0, i, 0)),
            pl.BlockSpec((NCORE, BLK, 128), lambda i: (0, i, 0)),
            pl.BlockSpec((NCORE, BLK, 16), lambda i: (0, i, 0)),
            pl.BlockSpec((1, D1), lambda i: (0, 0)),
            pl.BlockSpec((D1, D2P), lambda i: (0, 0)),
            pl.BlockSpec((D2P, 16), lambda i: (0, 0)),
        ],
        out_specs=[
            pl.BlockSpec((BLK, D2P), lambda i: (i, 0)),
            pl.BlockSpec((BLK, 16), lambda i: (i, 0)),
        ],
        out_shape=[
            jax.ShapeDtypeStruct((n_pad, D2P), F32),
            jax.ShapeDtypeStruct((n_pad, 16), F32),
        ],
    )(plo, phi, asum_p, b1_2d, w2_p, a2_comb_m)

    asum2, p2 = _make_feat2_kernel(n_pad, nb, nb_pad)(src_p, dst_p, a2, xw2)

    out = pl.pallas_call(
        _fin_body,
        grid=grid,
        in_specs=[
            pl.BlockSpec((NCORE, BLK, D2P), lambda i: (0, i, 0)),
            pl.BlockSpec((NCORE, BLK, 16), lambda i: (0, i, 0)),
            pl.BlockSpec((1, NC), lambda i: (0, 0)),
        ],
        out_specs=pl.BlockSpec((BLK, NC), lambda i: (i, 0)),
        out_shape=jax.ShapeDtypeStruct((n_pad, NC), F32),
    )(p2, asum2, b2_2d)

    return out[:N]


# speedup vs baseline: 2.9721x; 2.9721x over previous
import functools

import jax
import jax.numpy as jnp
from jax import lax
from jax.experimental import pallas as pl
from jax.experimental.pallas import tpu as pltpu
from jax.experimental.pallas import tpu_sc as plsc

F32 = jnp.float32
LANES = 16
NCORE = 2
NSUB = 16
NW = NCORE * NSUB
BATCH = 128

_SC_PARAMS = pltpu.CompilerParams(use_tc_tiling_on_sc=False)


def _leaky_relu(v):
    return jnp.where(v >= 0, v, 0.2 * v)


def _w_from_rows(gs_i, gd_i):
    return jnp.exp(_leaky_relu(gs_i + jnp.flip(gd_i)))


def _mm1_body(x_ref, w1_ref, ac_ref, xwlo_ref, xwhi_ref, a1_ref):
    xw = jnp.dot(x_ref[...], w1_ref[...], preferred_element_type=F32)
    xwlo_ref[...] = xw[:, :128]
    xwhi_ref[...] = xw[:, 128:]
    a1_ref[...] = jnp.dot(xw, ac_ref[...], preferred_element_type=F32)


def _mid_body(plo_ref, phi_ref, asum_ref, b1_ref, w2_ref, a2c_ref,
              xw2_ref, a2_ref):
    lo = plo_ref[0] + plo_ref[1]
    hi = phi_ref[0] + phi_ref[1]
    s = asum_ref[0] + asum_ref[1]
    s = jnp.where(s == 0, 1.0, s)
    blk = lo.shape[0]
    den_lo = jnp.broadcast_to(s[:, 0:4, None], (blk, 4, 32)).reshape(blk, 128)
    den_hi = jnp.broadcast_to(s[:, 4:8, None], (blk, 4, 32)).reshape(blk, 128)
    h = jnp.concatenate([lo / den_lo, hi / den_hi], axis=1) + b1_ref[...]
    h = jnp.where(h > 0, h, jnp.exp(jnp.minimum(h, 0.0)) - 1.0)
    xw2 = jnp.dot(h, w2_ref[...], preferred_element_type=F32)
    xw2_ref[...] = xw2
    a2_ref[...] = jnp.dot(xw2, a2c_ref[...], preferred_element_type=F32)


def _fin_body(p2_ref, asum2_ref, b2_ref, out_ref):
    o = p2_ref[0] + p2_ref[1]
    s = asum2_ref[0, :, 0:1] + asum2_ref[1, :, 0:1]
    s = jnp.where(s == 0, 1.0, s)
    o = o[:, :40] / s + b2_ref[...]
    m = jnp.max(o, axis=1, keepdims=True)
    lse = jnp.log(jnp.sum(jnp.exp(o - m), axis=1, keepdims=True))
    out_ref[...] = o - m - lse


def _zero_rows(buf, nrow, ncol16):
    zero = jnp.zeros((LANES,), F32)

    def body(i, _):
        for k in range(ncol16):
            buf[i, pl.ds(LANES * k, LANES)] = zero
        return 0

    lax.fori_loop(0, nrow, body, 0)


def _make_asum_kernel(n_pad, nb, nb_pad):
    rows_per_tile = n_pad // NSUB

    @functools.partial(
        pl.kernel,
        out_type=jax.ShapeDtypeStruct((NCORE, n_pad, 16), F32),
        mesh=plsc.VectorSubcoreMesh(core_axis_name="c", subcore_axis_name="s"),
        compiler_params=_SC_PARAMS,
        scratch_types=[
            pltpu.VMEM((nb_pad, BATCH), jnp.int32),
            pltpu.VMEM((nb_pad, BATCH), jnp.int32),
            pltpu.VMEM((BATCH, 16), F32),
            pltpu.VMEM((BATCH, 16), F32),
            pltpu.VMEM((BATCH, 16), F32),
            pltpu.VMEM_SHARED((n_pad, 16), F32),
            pltpu.SemaphoreType.DMA,
            pltpu.SemaphoreType.DMA,
        ],
    )
    def asum_k(src_hbm, dst_hbm, a_hbm, asum_hbm,
               src_v, dst_v, gs, gd, wb, accs, sem0, sem1):
        c = lax.axis_index("c")
        s = lax.axis_index("s")
        wid = s * NCORE + c
        row0 = s * rows_per_tile
        rows = pl.ds(row0, rows_per_tile)

        pltpu.sync_copy(src_hbm.at[wid], src_v)
        pltpu.sync_copy(dst_hbm.at[wid], dst_v)
        _zero_rows(gs, BATCH, 1)
        for i in range(rows_per_tile // BATCH):
            pltpu.sync_copy(gs, accs.at[pl.ds(row0 + i * BATCH, BATCH)])
        plsc.subcore_barrier()

        def batch_body(b, _):
            cp0 = pltpu.async_copy(a_hbm.at[src_v.at[b]], gs, sem0)
            cp1 = pltpu.async_copy(a_hbm.at[dst_v.at[b]], gd, sem1)
            cp0.wait()
            cp1.wait()

            def wloop(i, _):
                wb[i, :] = _w_from_rows(gs[i, :], gd[i, :])
                return 0

            lax.fori_loop(0, BATCH, wloop, 0)
            pltpu.sync_copy(wb, accs.at[dst_v.at[b]], add=True)
            return 0

        lax.fori_loop(0, nb, batch_body, 0)
        plsc.subcore_barrier()
        pltpu.sync_copy(accs.at[rows], asum_hbm.at[c, rows])

    return asum_k


def _make_feat_kernel(n_pad, nb, nb_pad, two_pass):
    rows_per_tile = n_pad // NSUB
    n_tables = 2 if two_pass else 1

    out_types = [
        jax.ShapeDtypeStruct((NCORE, n_pad, 128), F32)
        for _ in range(n_tables)]
    if two_pass:
        out_types.append(jax.ShapeDtypeStruct((NW, nb * BATCH, 16), F32))

    @functools.partial(
        pl.kernel,
        out_type=tuple(out_types),
        mesh=plsc.VectorSubcoreMesh(core_axis_name="c", subcore_axis_name="s"),
        compiler_params=_SC_PARAMS,
        scratch_types=[
            pltpu.VMEM((nb_pad, BATCH), jnp.int32),
            pltpu.VMEM((nb_pad, BATCH), jnp.int32),
            pltpu.VMEM((BATCH, 16), F32),
            pltpu.VMEM((BATCH, 16), F32),
            pltpu.VMEM((BATCH, 16), F32),
            pltpu.VMEM((BATCH, 128), F32),
            pltpu.VMEM_SHARED((n_pad, 128), F32),
            pltpu.SemaphoreType.DMA,
            pltpu.SemaphoreType.DMA,
            pltpu.SemaphoreType.DMA,
        ],
    )
    def feat_k(src_hbm, dst_hbm, a_hbm, *rest):
        xw_hbms = rest[:n_tables]
        out_hbms = rest[n_tables:2 * n_tables]
        rest = rest[2 * n_tables:]
        if two_pass:
            wdump_hbm, rest = rest[0], rest[1:]
        (src_v, dst_v, gs, gd, wb, xr, acc, sem0, sem1, sem2) = rest
        c = lax.axis_index("c")
        s = lax.axis_index("s")
        wid = s * NCORE + c
        row0 = s * rows_per_tile
        rows = pl.ds(row0, rows_per_tile)

        pltpu.sync_copy(src_hbm.at[wid], src_v)
        pltpu.sync_copy(dst_hbm.at[wid], dst_v)

        def zero_acc():
            _zero_rows(xr, BATCH, 8)
            for i in range(rows_per_tile // BATCH):
                pltpu.sync_copy(xr, acc.at[pl.ds(row0 + i * BATCH, BATCH)])

        def run_pass(xw_hbm, h0, reuse_w):
            def batch_body(b, _):
                cp2 = pltpu.async_copy(xw_hbm.at[src_v.at[b]], xr, sem2)
                if reuse_w:
                    pltpu.async_copy(
                        wdump_hbm.at[wid, pl.ds(b * BATCH, BATCH)], wb,
                        sem1).wait()
                else:
                    cp0 = pltpu.async_copy(a_hbm.at[src_v.at[b]], gs, sem0)
                    cp1 = pltpu.async_copy(a_hbm.at[dst_v.at[b]], gd, sem1)
                    cp0.wait()
                    cp1.wait()

                    def wloop(i, _):
                        wb[i, :] = _w_from_rows(gs[i, :], gd[i, :])
                        return 0

                    lax.fori_loop(0, BATCH, wloop, 0)
                    if two_pass:
                        pltpu.async_copy(
                            wb, wdump_hbm.at[wid, pl.ds(b * BATCH, BATCH)],
                            sem0).wait()
                cp2.wait()

                def mloop(e, _):
                    wvec = wb[e, :]
                    for v in range(8):
                        wv = wvec[h0 + (v // 2)]
                        xr[e, pl.ds(LANES * v, LANES)] = (
                            xr[e, pl.ds(LANES * v, LANES)] * wv)
                    return 0

                lax.fori_loop(0, BATCH, mloop, 0)
                pltpu.sync_copy(xr, acc.at[dst_v.at[b]], add=True)
                return 0

            lax.fori_loop(0, nb, batch_body, 0)

        for p in range(n_tables):
            zero_acc()
            plsc.subcore_barrier()
            run_pass(xw_hbms[p], 4 * p if two_pass else 0, reuse_w=p > 0)
            plsc.subcore_barrier()
            pltpu.sync_copy(acc.at[rows], out_hbms[p].at[c, rows])
            if p + 1 < n_tables:
                plsc.subcore_barrier()

    return feat_k


def kernel(x, edge_index, W1, att_src1, att_dst1, b1, W2, att_src2, att_dst2, b2):
    N, F = x.shape
    H1, C1 = att_src1.shape[1], att_src1.shape[2]
    D1 = H1 * C1
    NC = W2.shape[1]
    D2P = 128
    E = edge_index.shape[1]
    E2 = E + N
    n_pad = -(-(N + 1) // 2048) * 2048
    nb = -(-E2 // (NW * BATCH))
    nb_pad = -(-nb // 8) * 8
    e_pad = NW * nb * BATCH

    loop = jnp.arange(N, dtype=jnp.int32)
    src = jnp.concatenate([edge_index[0].astype(jnp.int32), loop])
    dst = jnp.concatenate([edge_index[1].astype(jnp.int32), loop])
    pad = jnp.full((e_pad - E2,), N, jnp.int32)
    src_p = jnp.pad(jnp.concatenate([src, pad]).reshape(NW, nb, BATCH),
                    ((0, 0), (0, nb_pad - nb), (0, 0)), constant_values=N)
    dst_p = jnp.pad(jnp.concatenate([dst, pad]).reshape(NW, nb, BATCH),
                    ((0, 0), (0, nb_pad - nb), (0, 0)), constant_values=N)

    x_pad = jnp.pad(x, ((0, n_pad - N), (0, 0)))
    eye1 = jnp.eye(H1, dtype=F32)
    a_src_m = (att_src1[0][:, :, None] * eye1[:, None, :]).reshape(D1, H1)
    a_dst_m = (att_dst1[0][:, :, None] * eye1[:, None, :]).reshape(D1, H1)
    a_comb_m = jnp.concatenate([a_src_m, a_dst_m[:, ::-1]], axis=1)
    w2_p = jnp.pad(W2, ((0, 0), (0, D2P - NC)))
    a2s_m = jnp.pad(jnp.tile(att_src2[0, 0][:, None], (1, 8)),
                    ((0, D2P - NC), (0, 0)))
    a2d_m = jnp.pad(jnp.tile(att_dst2[0, 0][:, None], (1, 8)),
                    ((0, D2P - NC), (0, 0)))
    a2_comb_m = jnp.concatenate([a2s_m, a2d_m], axis=1)
    b1_2d = b1[None, :]
    b2_2d = b2[None, :]

    BLK = 512
    grid = (n_pad // BLK,)

    xwlo, xwhi, a1 = pl.pallas_call(
        _mm1_body,
        grid=grid,
        in_specs=[
            pl.BlockSpec((BLK, F), lambda i: (i, 0)),
            pl.BlockSpec((F, D1), lambda i: (0, 0)),
            pl.BlockSpec((D1, 16), lambda i: (0, 0)),
        ],
        out_specs=[
            pl.BlockSpec((BLK, 128), lambda i: (i, 0)),
            pl.BlockSpec((BLK, 128), lambda i: (i, 0)),
            pl.BlockSpec((BLK, 16), lambda i: (i, 0)),
        ],
        out_shape=[
            jax.ShapeDtypeStruct((n_pad, 128), F32),
            jax.ShapeDtypeStruct((n_pad, 128), F32),
            jax.ShapeDtypeStruct((n_pad, 16), F32),
        ],
    )(x_pad, W1, a_comb_m)

    asum_p = _make_asum_kernel(n_pad, nb, nb_pad)(src_p, dst_p, a1)
    plo, phi, _ = _make_feat_kernel(n_pad, nb, nb_pad, True)(
        src_p, dst_p, a1, xwlo, xwhi)

    xw2, a2 = pl.pallas_call(
        _mid_body,
        grid=grid,
        in_specs=[
            pl.BlockSpec((NCORE, BLK, 128), lambda i: (0, i, 0)),
            pl.BlockSpec((NCORE, BLK, 128), lambda i: (0, i, 0)),
            pl.BlockSpec((NCORE, BLK, 16), lambda i: (0, i, 0)),
            pl.BlockSpec((1, D1), lambda i: (0, 0)),
            pl.BlockSpec((D1, D2P), lambda i: (0, 0)),
            pl.BlockSpec((D2P, 16), lambda i: (0, 0)),
        ],
        out_specs=[
            pl.BlockSpec((BLK, D2P), lambda i: (i, 0)),
            pl.BlockSpec((BLK, 16), lambda i: (i, 0)),
        ],
        out_shape=[
            jax.ShapeDtypeStruct((n_pad, D2P), F32),
            jax.ShapeDtypeStruct((n_pad, 16), F32),
        ],
    )(plo, phi, asum_p, b1_2d, w2_p, a2_comb_m)

    asum2 = _make_asum_kernel(n_pad, nb, nb_pad)(src_p, dst_p, a2)
    (p2,) = _make_feat_kernel(n_pad, nb, nb_pad, False)(
        src_p, dst_p, a2, xw2)

    out = pl.pallas_call(
        _fin_body,
        grid=grid,
        in_specs=[
            pl.BlockSpec((NCORE, BLK, D2P), lambda i: (0, i, 0)),
            pl.BlockSpec((NCORE, BLK, 16), lambda i: (0, i, 0)),
            pl.BlockSpec((1, NC), lambda i: (0, 0)),
        ],
        out_specs=pl.BlockSpec((BLK, NC), lambda i: (i, 0)),
        out_shape=jax.ShapeDtypeStruct((n_pad, NC), F32),
    )(p2, asum2, b2_2d)

    return out[:N]


# speedup vs baseline: 3.1429x; 1.0575x over previous
import functools

import jax
import jax.numpy as jnp
from jax import lax
from jax.experimental import pallas as pl
from jax.experimental.pallas import tpu as pltpu
from jax.experimental.pallas import tpu_sc as plsc

F32 = jnp.float32
LANES = 16
NCORE = 2
NSUB = 16
NW = NCORE * NSUB
BATCH = 128

_SC_PARAMS = pltpu.CompilerParams(use_tc_tiling_on_sc=False)


def _leaky_relu(v):
    return jnp.where(v >= 0, v, 0.2 * v)


def _w_from_rows(gs_i, gd_i):
    return jnp.exp(_leaky_relu(gs_i + jnp.flip(gd_i)))


def _mm1_body(x_ref, w1_ref, ac_ref, xwlo_ref, xwhi_ref, a1_ref):
    xw = jnp.dot(x_ref[...], w1_ref[...], preferred_element_type=F32)
    xwlo_ref[...] = xw[:, :128]
    xwhi_ref[...] = xw[:, 128:]
    a1_ref[...] = jnp.dot(xw, ac_ref[...], preferred_element_type=F32)


def _mid_body(plo_ref, phi_ref, asum_ref, b1_ref, w2_ref, a2c_ref,
              xw2_ref, a2_ref):
    lo = plo_ref[0] + plo_ref[1]
    hi = phi_ref[0] + phi_ref[1]
    s = asum_ref[0] + asum_ref[1]
    s = jnp.where(s == 0, 1.0, s)
    blk = lo.shape[0]
    den_lo = jnp.broadcast_to(s[:, 0:4, None], (blk, 4, 32)).reshape(blk, 128)
    den_hi = jnp.broadcast_to(s[:, 4:8, None], (blk, 4, 32)).reshape(blk, 128)
    h = jnp.concatenate([lo / den_lo, hi / den_hi], axis=1) + b1_ref[...]
    h = jnp.where(h > 0, h, jnp.exp(jnp.minimum(h, 0.0)) - 1.0)
    xw2 = jnp.dot(h, w2_ref[...], preferred_element_type=F32)
    xw2_ref[...] = xw2
    a2_ref[...] = jnp.dot(xw2, a2c_ref[...], preferred_element_type=F32)


def _fin_body(p2_ref, asum2_ref, b2_ref, out_ref):
    o = p2_ref[0] + p2_ref[1]
    s = asum2_ref[0, :, 0:1] + asum2_ref[1, :, 0:1]
    s = jnp.where(s == 0, 1.0, s)
    o = o[:, :40] / s + b2_ref[...]
    m = jnp.max(o, axis=1, keepdims=True)
    lse = jnp.log(jnp.sum(jnp.exp(o - m), axis=1, keepdims=True))
    out_ref[...] = o - m - lse


def _zero_rows(buf, nrow, ncol16):
    zero = jnp.zeros((LANES,), F32)

    def body(i, _):
        for k in range(ncol16):
            buf[i, pl.ds(LANES * k, LANES)] = zero
        return 0

    lax.fori_loop(0, nrow, body, 0)


def _make_asum_kernel(n_pad, nb, nb_pad):
    rows_per_tile = n_pad // NSUB

    @functools.partial(
        pl.kernel,
        out_type=(
            jax.ShapeDtypeStruct((NCORE, n_pad, 16), F32),
            jax.ShapeDtypeStruct((NW, nb * BATCH, 16), F32),
        ),
        mesh=plsc.VectorSubcoreMesh(core_axis_name="c", subcore_axis_name="s"),
        compiler_params=_SC_PARAMS,
        scratch_types=[
            pltpu.VMEM((nb_pad, BATCH), jnp.int32),
            pltpu.VMEM((nb_pad, BATCH), jnp.int32),
            pltpu.VMEM((BATCH, 16), F32),
            pltpu.VMEM((BATCH, 16), F32),
            pltpu.VMEM((BATCH, 16), F32),
            pltpu.VMEM((BATCH, 16), F32),
            pltpu.VMEM((BATCH, 16), F32),
            pltpu.VMEM((BATCH, 16), F32),
            pltpu.VMEM_SHARED((n_pad, 16), F32),
            pltpu.SemaphoreType.DMA,
            pltpu.SemaphoreType.DMA,
            pltpu.SemaphoreType.DMA,
            pltpu.SemaphoreType.DMA,
        ],
    )
    def asum_k(src_hbm, dst_hbm, a_hbm, asum_hbm, wdump_hbm,
               src_v, dst_v, gs_a, gd_a, gs_b, gd_b, wb_a, wb_b, accs,
               sem0, sem1, sem2, sem3):
        c = lax.axis_index("c")
        s = lax.axis_index("s")
        wid = s * NCORE + c
        row0 = s * rows_per_tile
        rows = pl.ds(row0, rows_per_tile)

        pltpu.sync_copy(src_hbm.at[wid], src_v)
        pltpu.sync_copy(dst_hbm.at[wid], dst_v)
        _zero_rows(gs_a, BATCH, 1)
        for i in range(rows_per_tile // BATCH):
            pltpu.sync_copy(gs_a, accs.at[pl.ds(row0 + i * BATCH, BATCH)])
        plsc.subcore_barrier()

        def issue(b, gs, gd, ss, sd):
            cs = pltpu.async_copy(a_hbm.at[src_v.at[b]], gs, ss)
            cd = pltpu.async_copy(a_hbm.at[dst_v.at[b]], gd, sd)
            return cs, cd

        def consume(b, cps, gs, gd, wb, sw):
            cps[0].wait()
            cps[1].wait()

            def wloop(i, _):
                wb[i, :] = _w_from_rows(gs[i, :], gd[i, :])
                return 0

            lax.fori_loop(0, BATCH, wloop, 0)
            cw = pltpu.async_copy(
                wb, wdump_hbm.at[wid, pl.ds(b * BATCH, BATCH)], sw)
            pltpu.sync_copy(wb, accs.at[dst_v.at[b]], add=True)
            return cw

        def pair_body(k, carry):
            b = 2 * k
            cps_a = issue(b, gs_a, gd_a, sem0, sem1)
            cps_b = issue(b + 1, gs_b, gd_b, sem2, sem3)
            cw_a = consume(b, cps_a, gs_a, gd_a, wb_a, sem0)
            cw_b = consume(b + 1, cps_b, gs_b, gd_b, wb_b, sem1)
            cw_a.wait()
            cw_b.wait()
            return carry

        lax.fori_loop(0, nb // 2, pair_body, 0)
        if nb % 2:
            cps_a = issue(nb - 1, gs_a, gd_a, sem0, sem1)
            cw = consume(nb - 1, cps_a, gs_a, gd_a, wb_a, sem0)
            cw.wait()
        plsc.subcore_barrier()
        pltpu.sync_copy(accs.at[rows], asum_hbm.at[c, rows])

    return asum_k


def _make_feat_kernel(n_pad, nb, nb_pad, two_pass):
    rows_per_tile = n_pad // NSUB
    n_tables = 2 if two_pass else 1

    @functools.partial(
        pl.kernel,
        out_type=tuple(
            jax.ShapeDtypeStruct((NCORE, n_pad, 128), F32)
            for _ in range(n_tables)),
        mesh=plsc.VectorSubcoreMesh(core_axis_name="c", subcore_axis_name="s"),
        compiler_params=_SC_PARAMS,
        scratch_types=[
            pltpu.VMEM((nb_pad, BATCH), jnp.int32),
            pltpu.VMEM((nb_pad, BATCH), jnp.int32),
            pltpu.VMEM((BATCH, 16), F32),
            pltpu.VMEM((BATCH, 128), F32),
            pltpu.VMEM_SHARED((n_pad, 128), F32),
            pltpu.SemaphoreType.DMA,
            pltpu.SemaphoreType.DMA,
        ],
    )
    def feat_k(src_hbm, dst_hbm, wdump_hbm, *rest):
        xw_hbms = rest[:n_tables]
        out_hbms = rest[n_tables:2 * n_tables]
        (src_v, dst_v, wb, xr, acc, sem0, sem2) = rest[2 * n_tables:]
        c = lax.axis_index("c")
        s = lax.axis_index("s")
        wid = s * NCORE + c
        row0 = s * rows_per_tile
        rows = pl.ds(row0, rows_per_tile)

        pltpu.sync_copy(src_hbm.at[wid], src_v)
        pltpu.sync_copy(dst_hbm.at[wid], dst_v)

        def zero_acc():
            _zero_rows(xr, BATCH, 8)
            for i in range(rows_per_tile // BATCH):
                pltpu.sync_copy(xr, acc.at[pl.ds(row0 + i * BATCH, BATCH)])

        def run_pass(xw_hbm, h0):
            def batch_body(b, _):
                cp2 = pltpu.async_copy(xw_hbm.at[src_v.at[b]], xr, sem2)
                cpw = pltpu.async_copy(
                    wdump_hbm.at[wid, pl.ds(b * BATCH, BATCH)], wb, sem0)
                cpw.wait()
                cp2.wait()

                def mloop(e, _):
                    wvec = wb[e, :]
                    for v in range(8):
                        wv = wvec[h0 + (v // 2)]
                        xr[e, pl.ds(LANES * v, LANES)] = (
                            xr[e, pl.ds(LANES * v, LANES)] * wv)
                    return 0

                lax.fori_loop(0, BATCH, mloop, 0)
                pltpu.sync_copy(xr, acc.at[dst_v.at[b]], add=True)
                return 0

            lax.fori_loop(0, nb, batch_body, 0)

        for p in range(n_tables):
            zero_acc()
            plsc.subcore_barrier()
            run_pass(xw_hbms[p], 4 * p if two_pass else 0)
            plsc.subcore_barrier()
            pltpu.sync_copy(acc.at[rows], out_hbms[p].at[c, rows])
            if p + 1 < n_tables:
                plsc.subcore_barrier()

    return feat_k


def kernel(x, edge_index, W1, att_src1, att_dst1, b1, W2, att_src2, att_dst2, b2):
    N, F = x.shape
    H1, C1 = att_src1.shape[1], att_src1.shape[2]
    D1 = H1 * C1
    NC = W2.shape[1]
    D2P = 128
    E = edge_index.shape[1]
    E2 = E + N
    n_pad = -(-(N + 1) // 2048) * 2048
    nb = -(-E2 // (NW * BATCH))
    nb_pad = -(-nb // 8) * 8
    e_pad = NW * nb * BATCH

    loop = jnp.arange(N, dtype=jnp.int32)
    src = jnp.concatenate([edge_index[0].astype(jnp.int32), loop])
    dst = jnp.concatenate([edge_index[1].astype(jnp.int32), loop])
    pad = jnp.full((e_pad - E2,), N, jnp.int32)
    src_p = jnp.pad(jnp.concatenate([src, pad]).reshape(NW, nb, BATCH),
                    ((0, 0), (0, nb_pad - nb), (0, 0)), constant_values=N)
    dst_p = jnp.pad(jnp.concatenate([dst, pad]).reshape(NW, nb, BATCH),
                    ((0, 0), (0, nb_pad - nb), (0, 0)), constant_values=N)

    x_pad = jnp.pad(x, ((0, n_pad - N), (0, 0)))
    eye1 = jnp.eye(H1, dtype=F32)
    a_src_m = (att_src1[0][:, :, None] * eye1[:, None, :]).reshape(D1, H1)
    a_dst_m = (att_dst1[0][:, :, None] * eye1[:, None, :]).reshape(D1, H1)
    a_comb_m = jnp.concatenate([a_src_m, a_dst_m[:, ::-1]], axis=1)
    w2_p = jnp.pad(W2, ((0, 0), (0, D2P - NC)))
    a2s_m = jnp.pad(jnp.tile(att_src2[0, 0][:, None], (1, 8)),
                    ((0, D2P - NC), (0, 0)))
    a2d_m = jnp.pad(jnp.tile(att_dst2[0, 0][:, None], (1, 8)),
                    ((0, D2P - NC), (0, 0)))
    a2_comb_m = jnp.concatenate([a2s_m, a2d_m], axis=1)
    b1_2d = b1[None, :]
    b2_2d = b2[None, :]

    BLK = 512
    grid = (n_pad // BLK,)

    xwlo, xwhi, a1 = pl.pallas_call(
        _mm1_body,
        grid=grid,
        in_specs=[
            pl.BlockSpec((BLK, F), lambda i: (i, 0)),
            pl.BlockSpec((F, D1), lambda i: (0, 0)),
            pl.BlockSpec((D1, 16), lambda i: (0, 0)),
        ],
        out_specs=[
            pl.BlockSpec((BLK, 128), lambda i: (i, 0)),
            pl.BlockSpec((BLK, 128), lambda i: (i, 0)),
            pl.BlockSpec((BLK, 16), lambda i: (i, 0)),
        ],
        out_shape=[
            jax.ShapeDtypeStruct((n_pad, 128), F32),
            jax.ShapeDtypeStruct((n_pad, 128), F32),
            jax.ShapeDtypeStruct((n_pad, 16), F32),
        ],
    )(x_pad, W1, a_comb_m)

    asum_p, wd1 = _make_asum_kernel(n_pad, nb, nb_pad)(src_p, dst_p, a1)
    plo, phi = _make_feat_kernel(n_pad, nb, nb_pad, True)(
        src_p, dst_p, wd1, xwlo, xwhi)

    xw2, a2 = pl.pallas_call(
        _mid_body,
        grid=grid,
        in_specs=[
            pl.BlockSpec((NCORE, BLK, 128), lambda i: (0, i, 0)),
            pl.BlockSpec((NCORE, BLK, 128), lambda i: (0, i, 0)),
            pl.BlockSpec((NCORE, BLK, 16), lambda i: (0, i, 0)),
            pl.BlockSpec((1, D1), lambda i: (0, 0)),
            pl.BlockSpec((D1, D2P), lambda i: (0, 0)),
            pl.BlockSpec((D2P, 16), lambda i: (0, 0)),
        ],
        out_specs=[
            pl.BlockSpec((BLK, D2P), lambda i: (i, 0)),
            pl.BlockSpec((BLK, 16), lambda i: (i, 0)),
        ],
        out_shape=[
            jax.ShapeDtypeStruct((n_pad, D2P), F32),
            jax.ShapeDtypeStruct((n_pad, 16), F32),
        ],
    )(plo, phi, asum_p, b1_2d, w2_p, a2_comb_m)

    asum2, wd2 = _make_asum_kernel(n_pad, nb, nb_pad)(src_p, dst_p, a2)
    (p2,) = _make_feat_kernel(n_pad, nb, nb_pad, False)(
        src_p, dst_p, wd2, xw2)

    out = pl.pallas_call(
        _fin_body,
        grid=grid,
        in_specs=[
            pl.BlockSpec((NCORE, BLK, D2P), lambda i: (0, i, 0)),
            pl.BlockSpec((NCORE, BLK, 16), lambda i: (0, i, 0)),
            pl.BlockSpec((1, NC), lambda i: (0, 0)),
        ],
        out_specs=pl.BlockSpec((BLK, NC), lambda i: (i, 0)),
        out_shape=jax.ShapeDtypeStruct((n_pad, NC), F32),
    )(p2, asum2, b2_2d)

    return out[:N]
